# Initial kernel scaffold; baseline (speedup 1.0000x reference)
import jax
import jax.numpy as jnp
from jax.experimental import pallas as pl


def kernel(x, table):
    raise NotImplementedError("write your pallas kernel here")


# speedup vs baseline: 3.9545x; 3.9545x over previous
import jax
import jax.numpy as jnp
from jax import lax
from jax.experimental import pallas as pl
from jax.experimental.pallas import tpu as pltpu
from jax.experimental.pallas import tpu_sc as plsc

VOCAB_PAD = 1024
EMB = 64
SCALE = 8.0
NC = 2
NS = 16
NW = NC * NS
B_TOTAL = 4096 * 200
B_PER_W = B_TOTAL // NW
CHUNK = 512
SUB = 128
N_CHUNKS = B_PER_W // CHUNK
ROWS_PER_TILE = VOCAB_PAD // NS


def _body(x_hbm, tab_hbm, out_hbm, scaled_hbm, tbuf, idxbuf, rows, sem):
    c = lax.axis_index("c")
    s = lax.axis_index("s")
    wid = s * NC + c

    pltpu.sync_copy(tab_hbm.at[pl.ds(s * ROWS_PER_TILE, ROWS_PER_TILE)], tbuf)

    def scale_row(r, carry):
        for j in range(EMB // 16):
            tbuf[r, pl.ds(j * 16, 16)] = tbuf[r, pl.ds(j * 16, 16)] * SCALE
        return carry

    lax.fori_loop(0, ROWS_PER_TILE, scale_row, 0)
    pltpu.sync_copy(
        tbuf, scaled_hbm.at[pl.ds(c * VOCAB_PAD + s * ROWS_PER_TILE, ROWS_PER_TILE)]
    )
    plsc.subcore_barrier()

    base = wid * B_PER_W
    coff = c * VOCAB_PAD

    def chunk(g, carry):
        off = base + g * CHUNK
        pltpu.sync_copy(x_hbm.at[pl.ds(off, CHUNK)], idxbuf)

        def addoff(i, cy):
            idxbuf[pl.ds(i * 16, 16)] = idxbuf[pl.ds(i * 16, 16)] + coff
            return cy

        lax.fori_loop(0, CHUNK // 16, addoff, 0)
        for k in range(CHUNK // SUB):
            pltpu.async_copy(
                scaled_hbm.at[idxbuf.at[pl.ds(k * SUB, SUB)]],
                rows.at[pl.ds(k * SUB, SUB)],
                sem,
            )
        pltpu.make_async_copy(scaled_hbm.at[idxbuf], rows, sem).wait()
        pltpu.sync_copy(rows, out_hbm.at[pl.ds(off, CHUNK)])
        return carry

    lax.fori_loop(0, N_CHUNKS, chunk, 0)


_sc_call = pl.kernel(
    _body,
    out_type=(
        jax.ShapeDtypeStruct((B_TOTAL, EMB), jnp.float32),
        jax.ShapeDtypeStruct((NC * VOCAB_PAD, EMB), jnp.float32),
    ),
    mesh=plsc.VectorSubcoreMesh(
        core_axis_name="c", subcore_axis_name="s", num_cores=NC, num_subcores=NS
    ),
    scratch_types=[
        pltpu.VMEM((ROWS_PER_TILE, EMB), jnp.float32),
        pltpu.VMEM((CHUNK,), jnp.int32),
        pltpu.VMEM((CHUNK, EMB), jnp.float32),
        pltpu.SemaphoreType.DMA,
    ],
    compiler_params=pltpu.CompilerParams(use_tc_tiling_on_sc=False),
)


def kernel(x, table):
    tab = jnp.pad(table, ((0, VOCAB_PAD - table.shape[0]), (0, 0)))
    out, _ = _sc_call(x.reshape(-1), tab)
    return out.reshape(x.shape[0], x.shape[1], EMB)


# speedup vs baseline: 4.9956x; 1.2633x over previous
import jax
import jax.numpy as jnp
from jax import lax
from jax.experimental import pallas as pl
from jax.experimental.pallas import tpu as pltpu
from jax.experimental.pallas import tpu_sc as plsc

VOCAB_PAD = 1024
EMB = 64
SCALE = 8.0
NC = 2
NS = 16
NW = NC * NS
B_TOTAL = 4096 * 200
B_PER_W = B_TOTAL // NW
CHUNK = 512
SUB = 128
N_CHUNKS = B_PER_W // CHUNK
ROWS_PER_TILE = VOCAB_PAD // NS


def _body(x_hbm, tab_hbm, out_hbm, shared, tbuf, idxbuf, rows0, rows1,
          gsem0, gsem1, osem0, osem1):
    s = lax.axis_index("s")
    wid = s * NC + lax.axis_index("c")
    rows = (rows0, rows1)
    gsem = (gsem0, gsem1)
    osem = (osem0, osem1)

    pltpu.sync_copy(tab_hbm.at[pl.ds(s * ROWS_PER_TILE, ROWS_PER_TILE)], tbuf)

    def scale_row(r, carry):
        for j in range(EMB // 16):
            tbuf[r, pl.ds(j * 16, 16)] = tbuf[r, pl.ds(j * 16, 16)] * SCALE
        return carry

    lax.fori_loop(0, ROWS_PER_TILE, scale_row, 0)
    pltpu.sync_copy(tbuf, shared.at[pl.ds(s * ROWS_PER_TILE, ROWS_PER_TILE)])

    base = wid * B_PER_W
    pltpu.sync_copy(x_hbm.at[pl.ds(base, B_PER_W)], idxbuf)
    plsc.subcore_barrier()

    def issue_gather(g, b):
        for k in range(CHUNK // SUB):
            pltpu.async_copy(
                shared.at[idxbuf.at[pl.ds(g * CHUNK + k * SUB, SUB)]],
                rows[b].at[pl.ds(k * SUB, SUB)],
                gsem[b],
            )

    def wait_chunk(sem, b):
        pltpu.make_async_copy(out_hbm.at[pl.ds(0, CHUNK)], rows[b], sem).wait()

    issue_gather(0, 0)

    def pair(gg, carry):
        for b in range(2):
            bp = 1 - b
            g = gg * 2 + b
            wait_chunk(gsem[b], b)

            @pl.when(g + 1 < N_CHUNKS)
            def _():
                @pl.when(g >= 1)
                def _():
                    wait_chunk(osem[bp], bp)
                issue_gather(g + 1, bp)

            pltpu.async_copy(rows[b], out_hbm.at[pl.ds(base + g * CHUNK, CHUNK)],
                             osem[b])
        return carry

    lax.fori_loop(0, N_CHUNKS // 2, pair, 0)
    wait_chunk(osem[0], 0)
    wait_chunk(osem[1], 1)


_sc_call = pl.kernel(
    _body,
    out_type=jax.ShapeDtypeStruct((B_TOTAL, EMB), jnp.float32),
    mesh=plsc.VectorSubcoreMesh(
        core_axis_name="c", subcore_axis_name="s", num_cores=NC, num_subcores=NS
    ),
    scratch_types=[
        pltpu.VMEM_SHARED((VOCAB_PAD, EMB), jnp.float32),
        pltpu.VMEM((ROWS_PER_TILE, EMB), jnp.float32),
        pltpu.VMEM((B_PER_W,), jnp.int32),
        pltpu.VMEM((CHUNK, EMB), jnp.float32),
        pltpu.VMEM((CHUNK, EMB), jnp.float32),
        pltpu.SemaphoreType.DMA,
        pltpu.SemaphoreType.DMA,
        pltpu.SemaphoreType.DMA,
        pltpu.SemaphoreType.DMA,
    ],
    compiler_params=pltpu.CompilerParams(use_tc_tiling_on_sc=False),
)


def kernel(x, table):
    tab = jnp.pad(table, ((0, VOCAB_PAD - table.shape[0]), (0, 0)))
    out = _sc_call(x.reshape(-1), tab)
    return out.reshape(x.shape[0], x.shape[1], EMB)
